# baseline (speedup 1.0000x reference)
import jax
import jax.numpy as jnp
from jax.experimental import pallas as pl
from jax.experimental.pallas import tpu as pltpu

M = 32
K = 256
D = 24
EMB = M * D


def _pq_kernel(x_ref, cb_ref, cbhi_ref, cblo_ref, quant_ref, codes_ref):
    x = x_ref[:]
    codes_cols = []
    quant_cols = []
    for m in range(M):
        cb_m = cb_ref[m]
        x_m = x[:, m * D:(m + 1) * D]
        scores = jax.lax.dot_general(x_m, cb_m, (((1,), (1,)), ((), ())))
        codes_m = jnp.argmax(scores, axis=1).astype(jnp.int32)
        onehot = (jax.lax.broadcasted_iota(jnp.int32, scores.shape, 1)
                  == codes_m[:, None]).astype(jnp.bfloat16)
        dnums = (((1,), (0,)), ((), ()))
        quant_m = (jax.lax.dot_general(onehot, cbhi_ref[m], dnums,
                                       preferred_element_type=jnp.float32)
                   + jax.lax.dot_general(onehot, cblo_ref[m], dnums,
                                         preferred_element_type=jnp.float32))
        codes_cols.append(codes_m[:, None])
        quant_cols.append(quant_m)
    quant_ref[:] = jnp.concatenate(quant_cols, axis=1)
    codes_ref[:] = jnp.concatenate(codes_cols, axis=1)


def kernel(x, codebook):
    B = x.shape[0]
    BT = 512
    grid = (B // BT,)
    cb_hi = codebook.astype(jnp.bfloat16)
    cb_lo = (codebook - cb_hi.astype(jnp.float32)).astype(jnp.bfloat16)
    quant, codes = pl.pallas_call(
        _pq_kernel,
        grid=grid,
        in_specs=[pl.BlockSpec((BT, EMB), lambda i: (i, 0)),
                  pl.BlockSpec((M, K, D), lambda i: (0, 0, 0)),
                  pl.BlockSpec((M, K, D), lambda i: (0, 0, 0)),
                  pl.BlockSpec((M, K, D), lambda i: (0, 0, 0))],
        out_specs=[pl.BlockSpec((BT, EMB), lambda i: (i, 0)),
                   pl.BlockSpec((BT, M), lambda i: (i, 0))],
        out_shape=(jax.ShapeDtypeStruct((B, EMB), jnp.float32),
                   jax.ShapeDtypeStruct((B, M), jnp.int32)),
        compiler_params=pltpu.CompilerParams(
            dimension_semantics=("parallel",)),
    )(x, codebook, cb_hi, cb_lo)
    return quant, codes


# speedup vs baseline: 1.0140x; 1.0140x over previous
import jax
import jax.numpy as jnp
from jax.experimental import pallas as pl
from jax.experimental.pallas import tpu as pltpu

M = 32
K = 256
D = 24
EMB = M * D


def _pq_kernel(x_ref, cb_ref, cbhi_ref, cblo_ref, quant_ref, codes_ref):
    x = x_ref[:]
    codes_cols = []
    quant_cols = []
    for m in range(M):
        cb_m = cb_ref[m]
        x_m = x[:, m * D:(m + 1) * D]
        scores = jax.lax.dot_general(x_m, cb_m, (((1,), (1,)), ((), ())))
        codes_m = jnp.argmax(scores, axis=1).astype(jnp.int32)
        onehot = (jax.lax.broadcasted_iota(jnp.int32, scores.shape, 1)
                  == codes_m[:, None]).astype(jnp.float32)
        quant_m = jax.lax.dot_general(onehot, cb_m, (((1,), (0,)), ((), ())))
        codes_cols.append(codes_m[:, None])
        quant_cols.append(quant_m)
    quant_ref[:] = jnp.concatenate(quant_cols, axis=1)
    codes_ref[:] = jnp.concatenate(codes_cols, axis=1)


def kernel(x, codebook):
    B = x.shape[0]
    BT = 512
    grid = (B // BT,)
    cb_hi = codebook.astype(jnp.bfloat16)
    cb_lo = (codebook - cb_hi.astype(jnp.float32)).astype(jnp.bfloat16)
    quant, codes = pl.pallas_call(
        _pq_kernel,
        grid=grid,
        in_specs=[pl.BlockSpec((BT, EMB), lambda i: (i, 0)),
                  pl.BlockSpec((M, K, D), lambda i: (0, 0, 0)),
                  pl.BlockSpec((M, K, D), lambda i: (0, 0, 0)),
                  pl.BlockSpec((M, K, D), lambda i: (0, 0, 0))],
        out_specs=[pl.BlockSpec((BT, EMB), lambda i: (i, 0)),
                   pl.BlockSpec((BT, M), lambda i: (i, 0))],
        out_shape=(jax.ShapeDtypeStruct((B, EMB), jnp.float32),
                   jax.ShapeDtypeStruct((B, M), jnp.int32)),
        compiler_params=pltpu.CompilerParams(
            dimension_semantics=("parallel",)),
    )(x, codebook, cb_hi, cb_lo)
    return quant, codes


# speedup vs baseline: 9.3308x; 9.2020x over previous
import jax
import jax.numpy as jnp
from jax.experimental import pallas as pl
from jax.experimental.pallas import tpu as pltpu

M = 32
K = 256
D = 24
EMB = M * D


def _pq_kernel(x_ref, cb_ref, cbt_ref, quantT_ref, codesT_ref):
    x = x_ref[:]
    bt = x.shape[0]
    codes_rows = []
    quant_rows = []
    for m in range(M):
        cb_m = cb_ref[m]
        cbt_m = cbt_ref[m]
        x_m = x[:, m * D:(m + 1) * D]
        scoresT = jax.lax.dot_general(cb_m, x_m, (((1,), (1,)), ((), ())))
        codes_m = jnp.argmax(scoresT, axis=0).astype(jnp.int32)
        idx = jnp.broadcast_to(codes_m[None, :], (D, bt))
        q0 = jnp.take_along_axis(cbt_m[:, :128], jnp.minimum(idx, 127), axis=1)
        q1 = jnp.take_along_axis(cbt_m[:, 128:], jnp.maximum(idx - 128, 0),
                                 axis=1)
        quant_rows.append(jnp.where(idx < 128, q0, q1))
        codes_rows.append(codes_m[None, :])
    quantT_ref[:] = jnp.concatenate(quant_rows, axis=0)
    codesT_ref[:] = jnp.concatenate(codes_rows, axis=0)


def kernel(x, codebook):
    B = x.shape[0]
    BT = 512
    grid = (B // BT,)
    cbt = codebook.transpose(0, 2, 1)
    quantT, codesT = pl.pallas_call(
        _pq_kernel,
        grid=grid,
        in_specs=[pl.BlockSpec((BT, EMB), lambda i: (i, 0)),
                  pl.BlockSpec((M, K, D), lambda i: (0, 0, 0)),
                  pl.BlockSpec((M, D, K), lambda i: (0, 0, 0))],
        out_specs=[pl.BlockSpec((EMB, BT), lambda i: (0, i)),
                   pl.BlockSpec((M, BT), lambda i: (0, i))],
        out_shape=(jax.ShapeDtypeStruct((EMB, B), jnp.float32),
                   jax.ShapeDtypeStruct((M, B), jnp.int32)),
        compiler_params=pltpu.CompilerParams(
            dimension_semantics=("parallel",)),
    )(x, codebook, cbt)
    return quantT.T, codesT.T
